# baseline (device time: 42689 ns/iter reference)
import jax
import jax.numpy as jnp
from jax import lax
from jax.experimental import pallas as pl
from jax.experimental.pallas import tpu as pltpu

N_DEV = 32
M_PER = 1024
K = 512
N_OUT = 512

M_STREAM = 256
STREAM_OPS = [
    ["x", "y4", "z4"],
    ["y4", "z4", "x"],
    ["z4", "x", "y4"],
    ["x", "z4", "y4"],
]
N_ROUNDS = 3
N_SEMS = 64

DO_RS = True
DO_AG = True


def kernel(t, W):
    def body(t_hbm, w_hbm, out_ref, t_ref, w_ref, acc_ref, rbuf_ref,
             send_sems, recv_sems, copy_sems):
        cp_t = pltpu.make_async_copy(t_hbm, t_ref, copy_sems.at[0])
        cp_t.start()
        cp_w = pltpu.make_async_copy(w_hbm, w_ref, copy_sems.at[1])
        cp_w.start()

        my = lax.axis_index("i")
        zc = my >> 3
        msub = my & 7
        yc = msub >> 1
        xc = (msub ^ yc) & 1

        def y_partner(cc):
            return 8 * zc + 2 * cc + (xc ^ (cc & 1))

        def z_partner(cc):
            return 8 * cc + msub

        partners = [my ^ 1]
        partners += [y_partner((yc + k) & 3) for k in range(1, 4)]
        partners += [z_partner((zc + k) & 3) for k in range(1, 4)]

        barrier_sem = pltpu.get_barrier_semaphore()
        for p in partners:
            pl.semaphore_signal(
                barrier_sem, inc=1,
                device_id=(p,), device_id_type=pl.DeviceIdType.MESH,
            )
        pl.semaphore_wait(barrier_sem, len(partners))
        cp_t.wait()

        ctr = [0]

        def make(src, dst, dev):
            i = ctr[0]
            ctr[0] += 1
            return pltpu.make_async_remote_copy(
                src_ref=src, dst_ref=dst,
                send_sem=send_sems.at[i], recv_sem=recv_sems.at[i],
                device_id=(dev,), device_id_type=pl.DeviceIdType.MESH,
            )

        n_s = len(STREAM_OPS)
        los = [jnp.int32(s * M_STREAM) for s in range(n_s)]
        ws = [M_STREAM] * n_s
        rb_offs = [s * 512 for s in range(n_s)]
        pend = [None] * n_s

        def issue_rs(s, r):
            src_buf = t_ref if r == 0 else acc_ref
            op = STREAM_OPS[s][r]
            if op == "x":
                half = ws[s] // 2
                send_lo = los[s] + (1 - xc) * half
                keep_lo = los[s] + xc * half
                if DO_RS:
                    rd = make(
                        src_buf.at[pl.ds(send_lo, half)],
                        rbuf_ref.at[pl.ds(rb_offs[s], half)],
                        my ^ 1,
                    )
                    rd.start()
                    pend[s] = ("rs", r, [rd], keep_lo, half, rb_offs[s], 1)
                los[s] = keep_lo
                ws[s] = half
                rb_offs[s] += half
            else:
                q = ws[s] // 4
                c = yc if op == "y4" else zc
                keep_lo = los[s] + c * q
                if DO_RS:
                    rds = []
                    for k in range(1, 4):
                        cc = (c + k) & 3
                        dev = y_partner(cc) if op == "y4" else z_partner(cc)
                        rd = make(
                            src_buf.at[pl.ds(los[s] + cc * q, q)],
                            rbuf_ref.at[pl.ds(rb_offs[s] + (k - 1) * q, q)],
                            dev,
                        )
                        rd.start()
                        rds.append(rd)
                    pend[s] = ("rs", r, rds, keep_lo, q, rb_offs[s], 3)
                los[s] = keep_lo
                ws[s] = q
                rb_offs[s] += 3 * q

        def issue_ag(s, j):
            op = STREAM_OPS[s][N_ROUNDS - 1 - j]
            w = ws[s]
            src = out_ref.at[pl.ds(los[s], w)]
            if op == "x":
                if DO_AG:
                    rd = make(src, out_ref.at[pl.ds(los[s], w)], my ^ 1)
                    rd.start()
                    pend[s] = ("ag", [rd], xc, 2)
                else:
                    pend[s] = ("agskip", xc, 2)
            else:
                c = yc if op == "y4" else zc
                if DO_AG:
                    rds = []
                    for k in range(1, 4):
                        cc = (c + k) & 3
                        dev = y_partner(cc) if op == "y4" else z_partner(cc)
                        rd = make(src, out_ref.at[pl.ds(los[s], w)], dev)
                        rd.start()
                        rds.append(rd)
                    pend[s] = ("ag", rds, c, 4)
                else:
                    pend[s] = ("agskip", c, 4)

        def finish(s):
            if pend[s] is None:
                return
            if pend[s][0] == "agskip":
                _, c, radix = pend[s]
                los[s] = los[s] - c * ws[s]
                ws[s] = ws[s] * radix
            elif pend[s][0] == "rs":
                _, r, rds, keep_lo, q, off, n_in = pend[s]
                for rd in rds:
                    rd.wait()
                base_buf = t_ref if r == 0 else acc_ref
                total = rbuf_ref[pl.ds(off, q), :]
                for j in range(1, n_in):
                    total = total + rbuf_ref[pl.ds(off + j * q, q), :]
                acc_ref[pl.ds(keep_lo, q), :] = (
                    base_buf[pl.ds(keep_lo, q), :] + total
                )
            else:
                _, rds, c, radix = pend[s]
                for rd in rds:
                    rd.wait()
                los[s] = los[s] - c * ws[s]
                ws[s] = ws[s] * radix
            pend[s] = None

        MM_BASE = 1984
        for step in range(2 * N_ROUNDS + 1):
            if step == N_ROUNDS:
                cp_w.wait()
                for s in range(n_s):
                    finish(s)
                    rbuf_ref[pl.ds(MM_BASE + 8 * s, 8), :] = (
                        acc_ref[pl.ds(los[s], 8), :]
                    )
                piece = jnp.dot(
                    rbuf_ref[pl.ds(MM_BASE, 32), :], w_ref[:, :],
                    preferred_element_type=jnp.float32,
                )
                for s in range(n_s):
                    out_ref[pl.ds(los[s], 8), :] = piece[8 * s:8 * (s + 1), :]
                continue
            for s in range(n_s):
                finish(s)
                if step < N_ROUNDS:
                    issue_rs(s, step)
                else:
                    issue_ag(s, step - N_ROUNDS - 1)
        for s in range(n_s):
            finish(s)

    return pl.pallas_call(
        body,
        out_shape=jax.ShapeDtypeStruct((M_PER, N_OUT), jnp.float32),
        in_specs=[
            pl.BlockSpec(memory_space=pltpu.MemorySpace.HBM),
            pl.BlockSpec(memory_space=pltpu.MemorySpace.HBM),
        ],
        out_specs=pl.BlockSpec(memory_space=pltpu.VMEM),
        scratch_shapes=[
            pltpu.VMEM((M_PER, K), jnp.float32),
            pltpu.VMEM((K, N_OUT), jnp.float32),
            pltpu.VMEM((M_PER, K), jnp.float32),
            pltpu.VMEM((4 * 512, K), jnp.float32),
            pltpu.SemaphoreType.DMA((N_SEMS,)),
            pltpu.SemaphoreType.DMA((N_SEMS,)),
            pltpu.SemaphoreType.DMA((2,)),
        ],
        compiler_params=pltpu.CompilerParams(collective_id=0),
    )(t, W)


# device time: 42456 ns/iter; 1.0055x vs baseline; 1.0055x over previous
import jax
import jax.numpy as jnp
from jax import lax
from jax.experimental import pallas as pl
from jax.experimental.pallas import tpu as pltpu

N_DEV = 32
M_PER = 1024
K = 512
N_OUT = 512

M_STREAM = 256
STREAM_OPS = [
    ["x", "y4", "z4"],
    ["y4", "z4", "x"],
    ["z4", "x", "y4"],
    ["x", "z4", "y4"],
]
N_ROUNDS = 3
N_SEMS = 64


def kernel(t, W):
    def body(t_ref, w_ref, out_ref, acc_ref, rbuf_ref, send_sems, recv_sems):
        my = lax.axis_index("i")
        zc = my >> 3
        msub = my & 7
        yc = msub >> 1
        xc = (msub ^ yc) & 1

        def y_partner(cc):
            return 8 * zc + 2 * cc + (xc ^ (cc & 1))

        def z_partner(cc):
            return 8 * cc + msub

        partners = [my ^ 1]
        partners += [y_partner((yc + k) & 3) for k in range(1, 4)]
        partners += [z_partner((zc + k) & 3) for k in range(1, 4)]

        barrier_sem = pltpu.get_barrier_semaphore()
        for p in partners:
            pl.semaphore_signal(
                barrier_sem, inc=1,
                device_id=(p,), device_id_type=pl.DeviceIdType.MESH,
            )
        pl.semaphore_wait(barrier_sem, len(partners))

        ctr = [0]

        def make(src, dst, dev):
            i = ctr[0]
            ctr[0] += 1
            return pltpu.make_async_remote_copy(
                src_ref=src, dst_ref=dst,
                send_sem=send_sems.at[i], recv_sem=recv_sems.at[i],
                device_id=(dev,), device_id_type=pl.DeviceIdType.MESH,
            )

        n_s = len(STREAM_OPS)
        los = [jnp.int32(s * M_STREAM) for s in range(n_s)]
        ws = [M_STREAM] * n_s
        rb_offs = [s * 512 for s in range(n_s)]
        pend = [None] * n_s

        def issue_rs(s, r):
            src_buf = t_ref if r == 0 else acc_ref
            op = STREAM_OPS[s][r]
            if op == "x":
                half = ws[s] // 2
                send_lo = los[s] + (1 - xc) * half
                keep_lo = los[s] + xc * half
                rd = make(
                    src_buf.at[pl.ds(send_lo, half)],
                    rbuf_ref.at[pl.ds(rb_offs[s], half)],
                    my ^ 1,
                )
                rd.start()
                pend[s] = ("rs", r, [rd], keep_lo, half, rb_offs[s], 1)
                los[s] = keep_lo
                ws[s] = half
                rb_offs[s] += half
            else:
                q = ws[s] // 4
                c = yc if op == "y4" else zc
                keep_lo = los[s] + c * q
                rds = []
                for k in range(1, 4):
                    cc = (c + k) & 3
                    dev = y_partner(cc) if op == "y4" else z_partner(cc)
                    rd = make(
                        src_buf.at[pl.ds(los[s] + cc * q, q)],
                        rbuf_ref.at[pl.ds(rb_offs[s] + (k - 1) * q, q)],
                        dev,
                    )
                    rd.start()
                    rds.append(rd)
                pend[s] = ("rs", r, rds, keep_lo, q, rb_offs[s], 3)
                los[s] = keep_lo
                ws[s] = q
                rb_offs[s] += 3 * q

        def issue_ag(s, j):
            op = STREAM_OPS[s][N_ROUNDS - 1 - j]
            w = ws[s]
            src = out_ref.at[pl.ds(los[s], w)]
            if op == "x":
                rd = make(src, out_ref.at[pl.ds(los[s], w)], my ^ 1)
                rd.start()
                pend[s] = ("ag", [rd], xc, 2)
            else:
                c = yc if op == "y4" else zc
                rds = []
                for k in range(1, 4):
                    cc = (c + k) & 3
                    dev = y_partner(cc) if op == "y4" else z_partner(cc)
                    rd = make(src, out_ref.at[pl.ds(los[s], w)], dev)
                    rd.start()
                    rds.append(rd)
                pend[s] = ("ag", rds, c, 4)

        def finish(s):
            if pend[s] is None:
                return
            if pend[s][0] == "rs":
                _, r, rds, keep_lo, q, off, n_in = pend[s]
                for rd in rds:
                    rd.wait()
                base_buf = t_ref if r == 0 else acc_ref
                total = rbuf_ref[pl.ds(off, q), :]
                for j in range(1, n_in):
                    total = total + rbuf_ref[pl.ds(off + j * q, q), :]
                acc_ref[pl.ds(keep_lo, q), :] = (
                    base_buf[pl.ds(keep_lo, q), :] + total
                )
            else:
                _, rds, c, radix = pend[s]
                for rd in rds:
                    rd.wait()
                los[s] = los[s] - c * ws[s]
                ws[s] = ws[s] * radix
            pend[s] = None

        MM_BASE = 1984
        for step in range(2 * N_ROUNDS + 1):
            if step == N_ROUNDS:
                for s in range(n_s):
                    finish(s)
                    rbuf_ref[pl.ds(MM_BASE + 8 * s, 8), :] = (
                        acc_ref[pl.ds(los[s], 8), :]
                    )
                piece = jnp.dot(
                    rbuf_ref[pl.ds(MM_BASE, 32), :], w_ref[:, :],
                    preferred_element_type=jnp.float32,
                )
                for s in range(n_s):
                    out_ref[pl.ds(los[s], 8), :] = piece[8 * s:8 * (s + 1), :]
                continue
            for s in range(n_s):
                finish(s)
                if step < N_ROUNDS:
                    issue_rs(s, step)
                else:
                    issue_ag(s, step - N_ROUNDS - 1)
        for s in range(n_s):
            finish(s)

    return pl.pallas_call(
        body,
        out_shape=jax.ShapeDtypeStruct((M_PER, N_OUT), jnp.float32),
        in_specs=[
            pl.BlockSpec(memory_space=pltpu.VMEM),
            pl.BlockSpec(memory_space=pltpu.VMEM),
        ],
        out_specs=pl.BlockSpec(memory_space=pltpu.VMEM),
        scratch_shapes=[
            pltpu.VMEM((M_PER, K), jnp.float32),
            pltpu.VMEM((4 * 512, K), jnp.float32),
            pltpu.SemaphoreType.DMA((N_SEMS,)),
            pltpu.SemaphoreType.DMA((N_SEMS,)),
        ],
        compiler_params=pltpu.CompilerParams(collective_id=0),
    )(t, W)


# device time: 40788 ns/iter; 1.0466x vs baseline; 1.0409x over previous
import jax
import jax.numpy as jnp
from jax import lax
from jax.experimental import pallas as pl
from jax.experimental.pallas import tpu as pltpu

N_DEV = 32
M_PER = 1024
K = 512
N_OUT = 512

M_STREAM = 256
C_HALF = 256
STREAM_OPS = [
    ["x", "y4", "z4"],
    ["y4", "z4", "x"],
    ["z4", "x", "y4"],
    ["x", "z4", "y4"],
]
N_ROUNDS = 3
N_SEMS = 128


def kernel(t, W):
    def body(t_ref, w_ref, out_ref, acc_ref, rbuf_ref, send_sems, recv_sems):
        my = lax.axis_index("i")
        zc = my >> 3
        msub = my & 7
        yc = msub >> 1
        xc = (msub ^ yc) & 1

        def y_partner(cc):
            return 8 * zc + 2 * cc + (xc ^ (cc & 1))

        def z_partner(cc):
            return 8 * cc + msub

        partners = [my ^ 1]
        partners += [y_partner((yc + k) & 3) for k in range(1, 4)]
        partners += [z_partner((zc + k) & 3) for k in range(1, 4)]

        barrier_sem = pltpu.get_barrier_semaphore()
        for p in partners:
            pl.semaphore_signal(
                barrier_sem, inc=1,
                device_id=(p,), device_id_type=pl.DeviceIdType.MESH,
            )
        pl.semaphore_wait(barrier_sem, len(partners))

        ctr = [0]

        def make(src, dst, dev):
            i = ctr[0]
            ctr[0] += 1
            return pltpu.make_async_remote_copy(
                src_ref=src, dst_ref=dst,
                send_sem=send_sems.at[i], recv_sem=recv_sems.at[i],
                device_id=(dev,), device_id_type=pl.DeviceIdType.MESH,
            )

        n_i = 2 * len(STREAM_OPS)
        los = [jnp.int32((i // 2) * M_STREAM) for i in range(n_i)]
        ws = [M_STREAM] * n_i
        rb_offs = [(i // 2) * 512 for i in range(n_i)]
        pend = [None] * n_i

        def cds(i):
            return pl.ds((i % 2) * C_HALF, C_HALF)

        def issue_rs(i, r):
            s = i // 2
            src_buf = t_ref if r == 0 else acc_ref
            op = STREAM_OPS[s][r]
            if op == "x":
                half = ws[i] // 2
                send_lo = los[i] + (1 - xc) * half
                keep_lo = los[i] + xc * half
                rd = make(
                    src_buf.at[pl.ds(send_lo, half), cds(i)],
                    rbuf_ref.at[pl.ds(rb_offs[i], half), cds(i)],
                    my ^ 1,
                )
                rd.start()
                pend[i] = ("rs", r, [rd], keep_lo, half, rb_offs[i], 1)
                los[i] = keep_lo
                ws[i] = half
                rb_offs[i] += half
            else:
                q = ws[i] // 4
                c = yc if op == "y4" else zc
                keep_lo = los[i] + c * q
                rds = []
                for k in range(1, 4):
                    cc = (c + k) & 3
                    dev = y_partner(cc) if op == "y4" else z_partner(cc)
                    rd = make(
                        src_buf.at[pl.ds(los[i] + cc * q, q), cds(i)],
                        rbuf_ref.at[pl.ds(rb_offs[i] + (k - 1) * q, q), cds(i)],
                        dev,
                    )
                    rd.start()
                    rds.append(rd)
                pend[i] = ("rs", r, rds, keep_lo, q, rb_offs[i], 3)
                los[i] = keep_lo
                ws[i] = q
                rb_offs[i] += 3 * q

        def issue_ag(i, j):
            s = i // 2
            op = STREAM_OPS[s][N_ROUNDS - 1 - j]
            w = ws[i]
            src = out_ref.at[pl.ds(los[i], w), cds(i)]
            if op == "x":
                rd = make(src, out_ref.at[pl.ds(los[i], w), cds(i)], my ^ 1)
                rd.start()
                pend[i] = ("ag", [rd], xc, 2)
            else:
                c = yc if op == "y4" else zc
                rds = []
                for k in range(1, 4):
                    cc = (c + k) & 3
                    dev = y_partner(cc) if op == "y4" else z_partner(cc)
                    rd = make(src, out_ref.at[pl.ds(los[i], w), cds(i)], dev)
                    rd.start()
                    rds.append(rd)
                pend[i] = ("ag", rds, c, 4)

        def finish(i):
            if pend[i] is None:
                return
            if pend[i][0] == "rs":
                _, r, rds, keep_lo, q, off, n_in = pend[i]
                for rd in rds:
                    rd.wait()
                base_buf = t_ref if r == 0 else acc_ref
                total = rbuf_ref[pl.ds(off, q), cds(i)]
                for j in range(1, n_in):
                    total = total + rbuf_ref[pl.ds(off + j * q, q), cds(i)]
                acc_ref[pl.ds(keep_lo, q), cds(i)] = (
                    base_buf[pl.ds(keep_lo, q), cds(i)] + total
                )
            else:
                _, rds, c, radix = pend[i]
                for rd in rds:
                    rd.wait()
                los[i] = los[i] - c * ws[i]
                ws[i] = ws[i] * radix
            pend[i] = None

        MM_BASE = 1984
        order = [1, 3, 5, 7, 0, 2, 4, 6]
        for step in range(2 * N_ROUNDS + 2):
            for i in order:
                h = i % 2
                idx = step - h
                finish(i)
                if i == 7 and step == N_ROUNDS + 1:
                    for s in range(len(STREAM_OPS)):
                        rbuf_ref[pl.ds(MM_BASE + 8 * s, 8), :] = (
                            acc_ref[pl.ds(los[2 * s], 8), :]
                        )
                    piece = jnp.dot(
                        rbuf_ref[pl.ds(MM_BASE, 32), :], w_ref[:, :],
                        preferred_element_type=jnp.float32,
                    )
                    for s in range(len(STREAM_OPS)):
                        out_ref[pl.ds(los[2 * s], 8), :] = (
                            piece[8 * s:8 * (s + 1), :]
                        )
                if 0 <= idx < N_ROUNDS:
                    issue_rs(i, idx)
                elif N_ROUNDS < idx <= 2 * N_ROUNDS:
                    issue_ag(i, idx - N_ROUNDS - 1)
        for i in order:
            finish(i)

    return pl.pallas_call(
        body,
        out_shape=jax.ShapeDtypeStruct((M_PER, N_OUT), jnp.float32),
        in_specs=[
            pl.BlockSpec(memory_space=pltpu.VMEM),
            pl.BlockSpec(memory_space=pltpu.VMEM),
        ],
        out_specs=pl.BlockSpec(memory_space=pltpu.VMEM),
        scratch_shapes=[
            pltpu.VMEM((M_PER, K), jnp.float32),
            pltpu.VMEM((2048, K), jnp.float32),
            pltpu.SemaphoreType.DMA((N_SEMS,)),
            pltpu.SemaphoreType.DMA((N_SEMS,)),
        ],
        compiler_params=pltpu.CompilerParams(collective_id=0),
    )(t, W)


# device time: 40724 ns/iter; 1.0483x vs baseline; 1.0016x over previous
import jax
import jax.numpy as jnp
from jax import lax
from jax.experimental import pallas as pl
from jax.experimental.pallas import tpu as pltpu

N_DEV = 32
M_PER = 1024
K = 512
N_OUT = 512

M_STREAM = 256
C_HALF = 256
STREAM_OPS = [
    ["x", "y4", "z4"],
    ["y4", "z4", "x"],
    ["z4", "x", "y4"],
    ["x", "z4", "y4"],
]
N_ROUNDS = 3
N_SEMS = 128


def kernel(t, W):
    def body(t_ref, w_ref, out_ref, acc_ref, rbuf_ref, send_sems, recv_sems):
        my = lax.axis_index("i")
        zc = my >> 3
        msub = my & 7
        yc = msub >> 1
        xc = (msub ^ yc) & 1

        def y_partner(cc):
            return 8 * zc + 2 * cc + (xc ^ (cc & 1))

        def z_partner(cc):
            return 8 * cc + msub

        partners = [my ^ 1]
        partners += [y_partner((yc + k) & 3) for k in range(1, 4)]
        partners += [z_partner((zc + k) & 3) for k in range(1, 4)]

        barrier_sem = pltpu.get_barrier_semaphore()
        for p in partners:
            pl.semaphore_signal(
                barrier_sem, inc=1,
                device_id=(p,), device_id_type=pl.DeviceIdType.MESH,
            )
        pl.semaphore_wait(barrier_sem, len(partners))

        ctr = [0]

        def make(src, dst, dev):
            i = ctr[0]
            ctr[0] += 1
            return pltpu.make_async_remote_copy(
                src_ref=src, dst_ref=dst,
                send_sem=send_sems.at[i], recv_sem=recv_sems.at[i],
                device_id=(dev,), device_id_type=pl.DeviceIdType.MESH,
            )

        n_i = 2 * len(STREAM_OPS)
        los = [jnp.int32((i // 2) * M_STREAM) for i in range(n_i)]
        ws = [M_STREAM] * n_i
        rb_offs = [(i // 2) * 512 for i in range(n_i)]
        pend = [None] * n_i

        def cds(i):
            return pl.ds((i % 2) * C_HALF, C_HALF)

        def issue_rs(i, r):
            s = i // 2
            src_buf = t_ref if r == 0 else acc_ref
            op = STREAM_OPS[s][r]
            if op == "x":
                half = ws[i] // 2
                send_lo = los[i] + (1 - xc) * half
                keep_lo = los[i] + xc * half
                rd = make(
                    src_buf.at[pl.ds(send_lo, half), cds(i)],
                    rbuf_ref.at[pl.ds(rb_offs[i], half), cds(i)],
                    my ^ 1,
                )
                rd.start()
                pend[i] = ("rs", r, [rd], keep_lo, half, rb_offs[i], 1)
                los[i] = keep_lo
                ws[i] = half
                rb_offs[i] += half
            else:
                q = ws[i] // 4
                c = yc if op == "y4" else zc
                keep_lo = los[i] + c * q
                rds = []
                for k in range(1, 4):
                    cc = (c + k) & 3
                    dev = y_partner(cc) if op == "y4" else z_partner(cc)
                    rd = make(
                        src_buf.at[pl.ds(los[i] + cc * q, q), cds(i)],
                        rbuf_ref.at[pl.ds(rb_offs[i] + (k - 1) * q, q), cds(i)],
                        dev,
                    )
                    rd.start()
                    rds.append(rd)
                pend[i] = ("rs", r, rds, keep_lo, q, rb_offs[i], 3)
                los[i] = keep_lo
                ws[i] = q
                rb_offs[i] += 3 * q

        def issue_ag(i, j):
            s = i // 2
            op = STREAM_OPS[s][N_ROUNDS - 1 - j]
            w = ws[i]
            src = out_ref.at[pl.ds(los[i], w), cds(i)]
            if op == "x":
                rd = make(src, out_ref.at[pl.ds(los[i], w), cds(i)], my ^ 1)
                rd.start()
                pend[i] = ("ag", [rd], xc, 2)
            else:
                c = yc if op == "y4" else zc
                rds = []
                for k in range(1, 4):
                    cc = (c + k) & 3
                    dev = y_partner(cc) if op == "y4" else z_partner(cc)
                    rd = make(src, out_ref.at[pl.ds(los[i], w), cds(i)], dev)
                    rd.start()
                    rds.append(rd)
                pend[i] = ("ag", rds, c, 4)

        def finish(i):
            if pend[i] is None:
                return
            if pend[i][0] == "rs":
                _, r, rds, keep_lo, q, off, n_in = pend[i]
                for rd in rds:
                    rd.wait()
                base_buf = t_ref if r == 0 else acc_ref
                total = rbuf_ref[pl.ds(off, q), cds(i)]
                for j in range(1, n_in):
                    total = total + rbuf_ref[pl.ds(off + j * q, q), cds(i)]
                acc_ref[pl.ds(keep_lo, q), cds(i)] = (
                    base_buf[pl.ds(keep_lo, q), cds(i)] + total
                )
            else:
                _, rds, c, radix = pend[i]
                for rd in rds:
                    rd.wait()
                los[i] = los[i] - c * ws[i]
                ws[i] = ws[i] * radix
            pend[i] = None

        MM_BASE = 1984
        order = [1, 3, 5, 7, 0, 2, 4, 6]
        piece0 = None
        for step in range(2 * N_ROUNDS + 2):
            for i in order:
                h = i % 2
                idx = step - h
                finish(i)
                if i == 7 and step == N_ROUNDS + 1:
                    for s in range(len(STREAM_OPS)):
                        rbuf_ref[pl.ds(MM_BASE + 8 * s, 8),
                                 pl.ds(C_HALF, C_HALF)] = (
                            acc_ref[pl.ds(los[2 * s], 8),
                                    pl.ds(C_HALF, C_HALF)]
                        )
                    piece = piece0 + jnp.dot(
                        rbuf_ref[pl.ds(MM_BASE, 32), pl.ds(C_HALF, C_HALF)],
                        w_ref[pl.ds(C_HALF, C_HALF), :],
                        preferred_element_type=jnp.float32,
                    )
                    for s in range(len(STREAM_OPS)):
                        out_ref[pl.ds(los[2 * s], 8), :] = (
                            piece[8 * s:8 * (s + 1), :]
                        )
                if 0 <= idx < N_ROUNDS:
                    issue_rs(i, idx)
                elif N_ROUNDS < idx <= 2 * N_ROUNDS:
                    issue_ag(i, idx - N_ROUNDS - 1)
            if step == N_ROUNDS:
                for s in range(len(STREAM_OPS)):
                    rbuf_ref[pl.ds(MM_BASE + 8 * s, 8), pl.ds(0, C_HALF)] = (
                        acc_ref[pl.ds(los[2 * s], 8), pl.ds(0, C_HALF)]
                    )
                piece0 = jnp.dot(
                    rbuf_ref[pl.ds(MM_BASE, 32), pl.ds(0, C_HALF)],
                    w_ref[pl.ds(0, C_HALF), :],
                    preferred_element_type=jnp.float32,
                )
        for i in order:
            finish(i)

    return pl.pallas_call(
        body,
        out_shape=jax.ShapeDtypeStruct((M_PER, N_OUT), jnp.float32),
        in_specs=[
            pl.BlockSpec(memory_space=pltpu.VMEM),
            pl.BlockSpec(memory_space=pltpu.VMEM),
        ],
        out_specs=pl.BlockSpec(memory_space=pltpu.VMEM),
        scratch_shapes=[
            pltpu.VMEM((M_PER, K), jnp.float32),
            pltpu.VMEM((2048, K), jnp.float32),
            pltpu.SemaphoreType.DMA((N_SEMS,)),
            pltpu.SemaphoreType.DMA((N_SEMS,)),
        ],
        compiler_params=pltpu.CompilerParams(collective_id=0),
    )(t, W)
